# baseline (speedup 1.0000x reference)
import functools
import math

import jax
import jax.numpy as jnp
from jax.experimental import pallas as pl
from jax.experimental.pallas import tpu as pltpu

D = 2048
H = 16
NOPE = 128
ROPE = 32
VH = 128
KV = 512
E = 8
TK = 2
MI = 512
SH = 2
INTER = 4096
VOC = 32000
L = 3
NDENSE = 1
B = 1
S = 2048
EPS = 1e-6

_F32 = jnp.float32


def _embed_kernel(ids_ref, emb_ref, o_ref):
    o_ref[...] = emb_ref[...]


def _embed(ids, emb):
    T = ids.shape[0]
    out = pl.pallas_call(
        _embed_kernel,
        grid_spec=pltpu.PrefetchScalarGridSpec(
            num_scalar_prefetch=1,
            grid=(T,),
            in_specs=[pl.BlockSpec((1, 1, D), lambda i, ids_ref: (ids_ref[i], 0, 0))],
            out_specs=pl.BlockSpec((1, 1, D), lambda i, ids_ref: (i, 0, 0)),
        ),
        out_shape=jax.ShapeDtypeStruct((T, 1, D), _F32),
    )(ids, emb.reshape(VOC, 1, D))
    return out.reshape(T, D)


def _rms_kernel(x_ref, g_ref, o_ref):
    x = x_ref[...]
    o_ref[...] = x * jax.lax.rsqrt(jnp.mean(x * x, axis=-1, keepdims=True) + EPS) * g_ref[...]


def _rms(x, g, bm=256):
    M, K = x.shape
    return pl.pallas_call(
        _rms_kernel,
        grid=(M // bm,),
        in_specs=[
            pl.BlockSpec((bm, K), lambda i: (i, 0)),
            pl.BlockSpec((1, K), lambda i: (0, 0)),
        ],
        out_specs=pl.BlockSpec((bm, K), lambda i: (i, 0)),
        out_shape=jax.ShapeDtypeStruct((M, K), _F32),
    )(x, g.reshape(1, K))


def _mm_kernel(x_ref, w_ref, o_ref):
    o_ref[...] = jnp.dot(x_ref[...], w_ref[...], preferred_element_type=_F32)


def _mm(x, w, bm=256, bn=256):
    M, K = x.shape
    _, N = w.shape
    if N % bn != 0:
        bn = N
    return pl.pallas_call(
        _mm_kernel,
        grid=(M // bm, N // bn),
        in_specs=[
            pl.BlockSpec((bm, K), lambda i, j: (i, 0)),
            pl.BlockSpec((K, bn), lambda i, j: (0, j)),
        ],
        out_specs=pl.BlockSpec((bm, bn), lambda i, j: (i, j)),
        out_shape=jax.ShapeDtypeStruct((M, N), _F32),
    )(x, w)


def _rope_kernel(x1_ref, x2_ref, c_ref, s_ref, o1_ref, o2_ref):
    c = c_ref[...]
    s = s_ref[...]
    x1 = x1_ref[0]
    x2 = x2_ref[0]
    o1_ref[0] = x1 * c - x2 * s
    o2_ref[0] = x1 * s + x2 * c


def _rope(x1, x2, cos, sin, bs=512):
    nh, T, half = x1.shape
    return pl.pallas_call(
        _rope_kernel,
        grid=(nh, T // bs),
        in_specs=[
            pl.BlockSpec((1, bs, half), lambda h, i: (h, i, 0)),
            pl.BlockSpec((1, bs, half), lambda h, i: (h, i, 0)),
            pl.BlockSpec((bs, half), lambda h, i: (i, 0)),
            pl.BlockSpec((bs, half), lambda h, i: (i, 0)),
        ],
        out_specs=[
            pl.BlockSpec((1, bs, half), lambda h, i: (h, i, 0)),
            pl.BlockSpec((1, bs, half), lambda h, i: (h, i, 0)),
        ],
        out_shape=[
            jax.ShapeDtypeStruct((nh, T, half), _F32),
            jax.ShapeDtypeStruct((nh, T, half), _F32),
        ],
    )(x1, x2, cos, sin)


def _flash_kernel(qn_ref, qp_ref, kn_ref, kp_ref, v_ref, o_ref, s_ref, *, bq, bk):
    i = pl.program_id(1)
    qn = qn_ref[0]
    qp = qp_ref[0]
    scale = math.sqrt(float(NOPE + ROPE))
    nblk = (i + 1) * bq // bk
    dn = (((1,), (1,)), ((), ()))

    def fill(j, _):
        kn = kn_ref[0, pl.ds(j * bk, bk), :]
        kp = kp_ref[pl.ds(j * bk, bk), :]
        s = jax.lax.dot_general(qn, kn, dn, preferred_element_type=_F32)
        s = s + jax.lax.dot_general(qp, kp, dn, preferred_element_type=_F32)
        s = s / scale
        row = i * bq + jax.lax.broadcasted_iota(jnp.int32, (bq, bk), 0)
        col = j * bk + jax.lax.broadcasted_iota(jnp.int32, (bq, bk), 1)
        s_ref[:, pl.ds(j * bk, bk)] = jnp.where(col <= row, s, -1e30)
        return 0

    jax.lax.fori_loop(0, nblk, fill, 0)

    nb_total = s_ref.shape[1] // bk

    def blank(j, _):
        s_ref[:, pl.ds(j * bk, bk)] = jnp.full((bq, bk), -1e30, _F32)
        return 0

    jax.lax.fori_loop(nblk, nb_total, blank, 0)

    s = s_ref[...]
    m = jnp.max(s, axis=-1, keepdims=True)
    p_un = jnp.exp(s - m)
    l = jnp.sum(p_un, axis=-1, keepdims=True)
    p = p_un / l
    o_ref[0] = jnp.dot(p, v_ref[0], preferred_element_type=_F32)


def _flash(qn, qp, kn, kp, v, bq=256, bk=256):
    nh, T, _ = qn.shape
    kfn = functools.partial(_flash_kernel, bq=bq, bk=bk)
    return pl.pallas_call(
        kfn,
        grid=(nh, T // bq),
        in_specs=[
            pl.BlockSpec((1, bq, NOPE), lambda h, i: (h, i, 0)),
            pl.BlockSpec((1, bq, ROPE), lambda h, i: (h, i, 0)),
            pl.BlockSpec((1, T, NOPE), lambda h, i: (h, 0, 0)),
            pl.BlockSpec((T, ROPE), lambda h, i: (0, 0)),
            pl.BlockSpec((1, T, VH), lambda h, i: (h, 0, 0)),
        ],
        out_specs=pl.BlockSpec((1, bq, VH), lambda h, i: (h, i, 0)),
        out_shape=jax.ShapeDtypeStruct((nh, T, VH), _F32),
        scratch_shapes=[pltpu.VMEM((bq, T), _F32)],
    )(qn, qp, kn, kp, v)


def _swiglu_kernel(x_ref, w1_ref, w3_ref, o_ref):
    x = x_ref[...]
    a = jnp.dot(x, w1_ref[...], preferred_element_type=_F32)
    b = jnp.dot(x, w3_ref[...], preferred_element_type=_F32)
    o_ref[...] = a * jax.nn.sigmoid(a) * b


def _swiglu_w_kernel(x_ref, w1_ref, w3_ref, c_ref, o_ref):
    x = x_ref[...]
    a = jnp.dot(x, w1_ref[...], preferred_element_type=_F32)
    b = jnp.dot(x, w3_ref[...], preferred_element_type=_F32)
    o_ref[...] = a * jax.nn.sigmoid(a) * b * c_ref[...]


def _swiglu(x, w1, w3, c=None, bm=256, bn=256):
    M, K = x.shape
    _, N = w1.shape
    if N % bn != 0:
        bn = N
    in_specs = [
        pl.BlockSpec((bm, K), lambda i, j: (i, 0)),
        pl.BlockSpec((K, bn), lambda i, j: (0, j)),
        pl.BlockSpec((K, bn), lambda i, j: (0, j)),
    ]
    args = [x, w1, w3]
    kfn = _swiglu_kernel
    if c is not None:
        in_specs.append(pl.BlockSpec((bm, 1), lambda i, j: (i, 0)))
        args.append(c)
        kfn = _swiglu_w_kernel
    return pl.pallas_call(
        kfn,
        grid=(M // bm, N // bn),
        in_specs=in_specs,
        out_specs=pl.BlockSpec((bm, bn), lambda i, j: (i, j)),
        out_shape=jax.ShapeDtypeStruct((M, N), _F32),
    )(*args)


def _router_kernel(x_ref, g_ref, o_ref):
    s = jnp.dot(x_ref[...], g_ref[...], preferred_element_type=_F32)
    s = jax.nn.softmax(s, axis=-1)
    iota = jax.lax.broadcasted_iota(jnp.int32, s.shape, 1)
    i1 = jnp.argmax(s, axis=-1)
    oh1 = iota == i1[:, None]
    m1 = jnp.max(s, axis=-1, keepdims=True)
    s2 = jnp.where(oh1, -jnp.inf, s)
    i2 = jnp.argmax(s2, axis=-1)
    oh2 = iota == i2[:, None]
    m2 = jnp.max(s2, axis=-1, keepdims=True)
    o_ref[...] = jnp.where(oh1, m1, 0.0) + jnp.where(oh2, m2, 0.0)


def _router(x, gate, bm=256):
    M, K = x.shape
    return pl.pallas_call(
        _router_kernel,
        grid=(M // bm,),
        in_specs=[
            pl.BlockSpec((bm, K), lambda i: (i, 0)),
            pl.BlockSpec((K, E), lambda i: (0, 0)),
        ],
        out_specs=pl.BlockSpec((bm, E), lambda i: (i, 0)),
        out_shape=jax.ShapeDtypeStruct((M, E), _F32),
    )(x, gate)


def _rope_tables():
    half = ROPE // 2
    pos = jnp.arange(S, dtype=jnp.float32)
    inv = 1.0 / (10000.0 ** (jnp.arange(half, dtype=jnp.float32) / half))
    ang = pos[:, None] * inv[None, :]
    return jnp.cos(ang), jnp.sin(ang)


def _attn_layer(p, x, cos, sin):
    h = _rms(x, p['attn_norm'])
    q = _mm(h, p['wq'])
    kv = _mm(h, p['wkv_a'])
    kv_c = _rms(kv[:, :KV], p['kv_norm'])
    kvb = _mm(kv_c, p['wkv_b'])

    q3 = q.reshape(S, H, NOPE + ROPE)
    qn = q3[..., :NOPE].transpose(1, 0, 2)
    qp_half = ROPE // 2
    q_pe = q3[..., NOPE:]
    qp1 = q_pe[..., :qp_half].transpose(1, 0, 2)
    qp2 = q_pe[..., qp_half:].transpose(1, 0, 2)
    qp1r, qp2r = _rope(qp1, qp2, cos, sin)
    qp = jnp.concatenate([qp1r, qp2r], axis=-1)

    k_pe = kv[:, KV:]
    kp1 = k_pe[:, :qp_half][None]
    kp2 = k_pe[:, qp_half:][None]
    kp1r, kp2r = _rope(kp1, kp2, cos, sin)
    kp = jnp.concatenate([kp1r, kp2r], axis=-1)[0]

    kvb3 = kvb.reshape(S, H, NOPE + VH)
    kn = kvb3[..., :NOPE].transpose(1, 0, 2)
    v = kvb3[..., NOPE:].transpose(1, 0, 2)

    o = _flash(qn, qp, kn, kp, v)
    o2 = o.transpose(1, 0, 2).reshape(S, H * VH)
    return _mm(o2, p['wo'])


def _moe_layer(p, hh):
    comb = _router(hh, p['gate'])
    g = _swiglu(hh, p['sw1'], p['sw3'])
    acc = _mm(g, p['sw2'])
    for e in range(E):
        ge = _swiglu(hh, p['ew1'][e], p['ew3'][e], c=comb[:, e:e + 1])
        acc = acc + _mm(ge, p['ew2'][e])
    return acc


def kernel(input_ids, params):
    ids = input_ids.reshape(B * S)
    x = _embed(ids, params['embed'])
    cos, sin = _rope_tables()
    for i in range(L):
        p = params['layer_%d' % i]
        x = x + _attn_layer(p, x, cos, sin)
        hh = _rms(x, p['ffn_norm'])
        if i < NDENSE:
            g = _swiglu(hh, p['w1'], p['w3'])
            x = x + _mm(g, p['w2'])
        else:
            x = x + _moe_layer(p, hh)
    h = _rms(x, params['final_norm'])
    logits = _mm(h, params['head'], bm=256, bn=640)
    return logits.reshape(B, S, VOC)


# speedup vs baseline: 1.9237x; 1.9237x over previous
import functools
import math

import jax
import jax.numpy as jnp
from jax.experimental import pallas as pl
from jax.experimental.pallas import tpu as pltpu

D = 2048
H = 16
NOPE = 128
ROPE = 32
VH = 128
KV = 512
E = 8
TK = 2
MI = 512
SH = 2
INTER = 4096
VOC = 32000
L = 3
NDENSE = 1
B = 1
S = 2048
EPS = 1e-6

_F32 = jnp.float32


def _embed_kernel(ids_ref, emb_ref, o_ref, sem, *, bm):
    i = pl.program_id(0)

    def start(r, _):
        idx = ids_ref[i * bm + r]
        pltpu.make_async_copy(emb_ref.at[idx], o_ref.at[r], sem).start()
        return 0

    jax.lax.fori_loop(0, bm, start, 0)

    def wait(r, _):
        pltpu.make_async_copy(emb_ref.at[0], o_ref.at[0], sem).wait()
        return 0

    jax.lax.fori_loop(0, bm, wait, 0)


def _embed(ids, emb, bm=256):
    T = ids.shape[0]
    return pl.pallas_call(
        functools.partial(_embed_kernel, bm=bm),
        grid_spec=pltpu.PrefetchScalarGridSpec(
            num_scalar_prefetch=1,
            grid=(T // bm,),
            in_specs=[pl.BlockSpec(memory_space=pltpu.MemorySpace.HBM)],
            out_specs=pl.BlockSpec((bm, D), lambda i, ids_ref: (i, 0)),
            scratch_shapes=[pltpu.SemaphoreType.DMA],
        ),
        out_shape=jax.ShapeDtypeStruct((T, D), _F32),
    )(ids, emb)


def _rms_kernel(x_ref, g_ref, o_ref):
    x = x_ref[...]
    o_ref[...] = x * jax.lax.rsqrt(jnp.mean(x * x, axis=-1, keepdims=True) + EPS) * g_ref[...]


def _rms(x, g, bm=256):
    M, K = x.shape
    return pl.pallas_call(
        _rms_kernel,
        grid=(M // bm,),
        in_specs=[
            pl.BlockSpec((bm, K), lambda i: (i, 0)),
            pl.BlockSpec((1, K), lambda i: (0, 0)),
        ],
        out_specs=pl.BlockSpec((bm, K), lambda i: (i, 0)),
        out_shape=jax.ShapeDtypeStruct((M, K), _F32),
    )(x, g.reshape(1, K))


def _mm_kernel(x_ref, w_ref, o_ref):
    o_ref[...] = jnp.dot(x_ref[...], w_ref[...], preferred_element_type=_F32)


def _mm(x, w, bm=2048, bn=512):
    M, K = x.shape
    _, N = w.shape
    if K > 2048:
        bm = min(bm, 1024)
    if N % bn != 0:
        bn = N
    return pl.pallas_call(
        _mm_kernel,
        grid=(M // bm, N // bn),
        in_specs=[
            pl.BlockSpec((bm, K), lambda i, j: (i, 0)),
            pl.BlockSpec((K, bn), lambda i, j: (0, j)),
        ],
        out_specs=pl.BlockSpec((bm, bn), lambda i, j: (i, j)),
        out_shape=jax.ShapeDtypeStruct((M, N), _F32),
    )(x, w)


def _rope_kernel(x1_ref, x2_ref, c_ref, s_ref, o1_ref, o2_ref):
    c = c_ref[...]
    s = s_ref[...]
    x1 = x1_ref[0]
    x2 = x2_ref[0]
    o1_ref[0] = x1 * c - x2 * s
    o2_ref[0] = x1 * s + x2 * c


def _rope(x1, x2, cos, sin, bs=512):
    nh, T, half = x1.shape
    return pl.pallas_call(
        _rope_kernel,
        grid=(nh, T // bs),
        in_specs=[
            pl.BlockSpec((1, bs, half), lambda h, i: (h, i, 0)),
            pl.BlockSpec((1, bs, half), lambda h, i: (h, i, 0)),
            pl.BlockSpec((bs, half), lambda h, i: (i, 0)),
            pl.BlockSpec((bs, half), lambda h, i: (i, 0)),
        ],
        out_specs=[
            pl.BlockSpec((1, bs, half), lambda h, i: (h, i, 0)),
            pl.BlockSpec((1, bs, half), lambda h, i: (h, i, 0)),
        ],
        out_shape=[
            jax.ShapeDtypeStruct((nh, T, half), _F32),
            jax.ShapeDtypeStruct((nh, T, half), _F32),
        ],
    )(x1, x2, cos, sin)


def _flash_kernel(qn_ref, qp_ref, kn_ref, kp_ref, v_ref, o_ref, s_ref, *, bq, bk):
    i = pl.program_id(1)
    qn = qn_ref[0]
    qp = qp_ref[0]
    scale = math.sqrt(float(NOPE + ROPE))
    nblk = (i + 1) * bq // bk
    dn = (((1,), (1,)), ((), ()))

    def fill(j, _):
        kn = kn_ref[0, pl.ds(j * bk, bk), :]
        kp = kp_ref[pl.ds(j * bk, bk), :]
        s = jax.lax.dot_general(qn, kn, dn, preferred_element_type=_F32)
        s = s + jax.lax.dot_general(qp, kp, dn, preferred_element_type=_F32)
        s = s / scale
        row = i * bq + jax.lax.broadcasted_iota(jnp.int32, (bq, bk), 0)
        col = j * bk + jax.lax.broadcasted_iota(jnp.int32, (bq, bk), 1)
        s_ref[:, pl.ds(j * bk, bk)] = jnp.where(col <= row, s, -1e30)
        return 0

    jax.lax.fori_loop(0, nblk, fill, 0)

    nb_total = s_ref.shape[1] // bk

    def blank(j, _):
        s_ref[:, pl.ds(j * bk, bk)] = jnp.full((bq, bk), -1e30, _F32)
        return 0

    jax.lax.fori_loop(nblk, nb_total, blank, 0)

    s = s_ref[...]
    m = jnp.max(s, axis=-1, keepdims=True)
    p_un = jnp.exp(s - m)
    l = jnp.sum(p_un, axis=-1, keepdims=True)
    p = p_un / l
    o_ref[0] = jnp.dot(p, v_ref[0], preferred_element_type=_F32)


def _flash(qn, qp, kn, kp, v, bq=256, bk=256):
    nh, T, _ = qn.shape
    kfn = functools.partial(_flash_kernel, bq=bq, bk=bk)
    return pl.pallas_call(
        kfn,
        grid=(nh, T // bq),
        in_specs=[
            pl.BlockSpec((1, bq, NOPE), lambda h, i: (h, i, 0)),
            pl.BlockSpec((1, bq, ROPE), lambda h, i: (h, i, 0)),
            pl.BlockSpec((1, T, NOPE), lambda h, i: (h, 0, 0)),
            pl.BlockSpec((T, ROPE), lambda h, i: (0, 0)),
            pl.BlockSpec((1, T, VH), lambda h, i: (h, 0, 0)),
        ],
        out_specs=pl.BlockSpec((1, bq, VH), lambda h, i: (h, i, 0)),
        out_shape=jax.ShapeDtypeStruct((nh, T, VH), _F32),
        scratch_shapes=[pltpu.VMEM((bq, T), _F32)],
    )(qn, qp, kn, kp, v)


def _swiglu_kernel(x_ref, w1_ref, w3_ref, o_ref):
    x = x_ref[...]
    a = jnp.dot(x, w1_ref[...], preferred_element_type=_F32)
    b = jnp.dot(x, w3_ref[...], preferred_element_type=_F32)
    o_ref[...] = a * jax.nn.sigmoid(a) * b


def _swiglu_w_kernel(x_ref, w1_ref, w3_ref, c_ref, o_ref):
    x = x_ref[...]
    a = jnp.dot(x, w1_ref[...], preferred_element_type=_F32)
    b = jnp.dot(x, w3_ref[...], preferred_element_type=_F32)
    o_ref[...] = a * jax.nn.sigmoid(a) * b * c_ref[...]


def _swiglu(x, w1, w3, c=None, bm=2048, bn=512):
    M, K = x.shape
    _, N = w1.shape
    if N % bn != 0:
        bn = N
    in_specs = [
        pl.BlockSpec((bm, K), lambda i, j: (i, 0)),
        pl.BlockSpec((K, bn), lambda i, j: (0, j)),
        pl.BlockSpec((K, bn), lambda i, j: (0, j)),
    ]
    args = [x, w1, w3]
    kfn = _swiglu_kernel
    if c is not None:
        in_specs.append(pl.BlockSpec((bm, 1), lambda i, j: (i, 0)))
        args.append(c)
        kfn = _swiglu_w_kernel
    return pl.pallas_call(
        kfn,
        grid=(M // bm, N // bn),
        in_specs=in_specs,
        out_specs=pl.BlockSpec((bm, bn), lambda i, j: (i, j)),
        out_shape=jax.ShapeDtypeStruct((M, N), _F32),
    )(*args)


def _router_kernel(x_ref, g_ref, o_ref):
    s = jnp.dot(x_ref[...], g_ref[...], preferred_element_type=_F32)
    s = jax.nn.softmax(s, axis=-1)
    iota = jax.lax.broadcasted_iota(jnp.int32, s.shape, 1)
    i1 = jnp.argmax(s, axis=-1)
    oh1 = iota == i1[:, None]
    m1 = jnp.max(s, axis=-1, keepdims=True)
    s2 = jnp.where(oh1, -jnp.inf, s)
    i2 = jnp.argmax(s2, axis=-1)
    oh2 = iota == i2[:, None]
    m2 = jnp.max(s2, axis=-1, keepdims=True)
    o_ref[...] = jnp.where(oh1, m1, 0.0) + jnp.where(oh2, m2, 0.0)


def _router(x, gate, bm=256):
    M, K = x.shape
    return pl.pallas_call(
        _router_kernel,
        grid=(M // bm,),
        in_specs=[
            pl.BlockSpec((bm, K), lambda i: (i, 0)),
            pl.BlockSpec((K, E), lambda i: (0, 0)),
        ],
        out_specs=pl.BlockSpec((bm, E), lambda i: (i, 0)),
        out_shape=jax.ShapeDtypeStruct((M, E), _F32),
    )(x, gate)


def _rope_tables():
    half = ROPE // 2
    pos = jnp.arange(S, dtype=jnp.float32)
    inv = 1.0 / (10000.0 ** (jnp.arange(half, dtype=jnp.float32) / half))
    ang = pos[:, None] * inv[None, :]
    return jnp.cos(ang), jnp.sin(ang)


def _attn_layer(p, x, cos, sin):
    h = _rms(x, p['attn_norm'])
    q = _mm(h, p['wq'])
    kv = _mm(h, p['wkv_a'])
    kv_c = _rms(kv[:, :KV], p['kv_norm'])
    kvb = _mm(kv_c, p['wkv_b'])

    q3 = q.reshape(S, H, NOPE + ROPE)
    qn = q3[..., :NOPE].transpose(1, 0, 2)
    qp_half = ROPE // 2
    q_pe = q3[..., NOPE:]
    qp1 = q_pe[..., :qp_half].transpose(1, 0, 2)
    qp2 = q_pe[..., qp_half:].transpose(1, 0, 2)
    qp1r, qp2r = _rope(qp1, qp2, cos, sin)
    qp = jnp.concatenate([qp1r, qp2r], axis=-1)

    k_pe = kv[:, KV:]
    kp1 = k_pe[:, :qp_half][None]
    kp2 = k_pe[:, qp_half:][None]
    kp1r, kp2r = _rope(kp1, kp2, cos, sin)
    kp = jnp.concatenate([kp1r, kp2r], axis=-1)[0]

    kvb3 = kvb.reshape(S, H, NOPE + VH)
    kn = kvb3[..., :NOPE].transpose(1, 0, 2)
    v = kvb3[..., NOPE:].transpose(1, 0, 2)

    o = _flash(qn, qp, kn, kp, v)
    o2 = o.transpose(1, 0, 2).reshape(S, H * VH)
    return _mm(o2, p['wo'])


def _moe_layer(p, hh):
    comb = _router(hh, p['gate'])
    g = _swiglu(hh, p['sw1'], p['sw3'])
    acc = _mm(g, p['sw2'])
    for e in range(E):
        ge = _swiglu(hh, p['ew1'][e], p['ew3'][e], c=comb[:, e:e + 1])
        acc = acc + _mm(ge, p['ew2'][e])
    return acc


def kernel(input_ids, params):
    ids = input_ids.reshape(B * S)
    x = _embed(ids, params['embed'])
    cos, sin = _rope_tables()
    for i in range(L):
        p = params['layer_%d' % i]
        x = x + _attn_layer(p, x, cos, sin)
        hh = _rms(x, p['ffn_norm'])
        if i < NDENSE:
            g = _swiglu(hh, p['w1'], p['w3'])
            x = x + _mm(g, p['w2'])
        else:
            x = x + _moe_layer(p, hh)
    h = _rms(x, params['final_norm'])
    logits = _mm(h, params['head'], bm=2048, bn=640)
    return logits.reshape(B, S, VOC)
